# baseline (speedup 1.0000x reference)
import functools

import jax
import jax.numpy as jnp
from jax import lax
from jax.experimental import pallas as pl
from jax.experimental.pallas import tpu as pltpu, tpu_sc as plsc

NS = 16
NC = 2
NW = NC * NS


def _make_sc_agg(n_rows_tbl, width, n_rows_acc, n_chunks, K, dtype):
  rpt = n_rows_acc // NS
  mesh = plsc.VectorSubcoreMesh(core_axis_name="c", subcore_axis_name="s")

  @functools.partial(
      pl.kernel,
      out_type=jax.ShapeDtypeStruct((NC, n_rows_acc, width), dtype),
      mesh=mesh,
      compiler_params=pltpu.CompilerParams(use_tc_tiling_on_sc=False),
      scratch_types=[
          pltpu.VMEM((n_chunks, K), jnp.int32),
          pltpu.VMEM((n_chunks, K), jnp.int32),
          pltpu.VMEM((2, K, width), dtype),
          pltpu.VMEM_SHARED((n_rows_acc, width), dtype),
          pltpu.SemaphoreType.DMA,
          pltpu.SemaphoreType.DMA,
      ],
  )
  def sc_agg(tbl_hbm, src_hbm, dst_hbm, zeros_hbm, out_hbm,
             src_v, dst_v, rows_v, acc_sh, sem_a, sem_b):
    c = lax.axis_index("c")
    s = lax.axis_index("s")
    wid = c * NS + s
    pltpu.sync_copy(zeros_hbm, acc_sh.at[pl.ds(s * rpt, rpt)])
    pltpu.sync_copy(src_hbm.at[wid], src_v)
    pltpu.sync_copy(dst_hbm.at[wid], dst_v)
    plsc.subcore_barrier()

    def gather(ci, buf, sem):
      return pltpu.make_async_copy(tbl_hbm.at[src_v.at[ci]],
                                   rows_v.at[buf], sem)

    def scatter(ci, buf):
      pltpu.sync_copy(rows_v.at[buf], acc_sh.at[dst_v.at[ci]], add=True)

    gather(0, 0, sem_a).start()

    def body(p, carry):
      ci = 2 * p

      @pl.when(ci + 1 < n_chunks)
      def _():
        gather(ci + 1, 1, sem_b).start()

      gather(ci, 0, sem_a).wait()
      scatter(ci, 0)

      @pl.when(ci + 2 < n_chunks)
      def _():
        gather(ci + 2, 0, sem_a).start()

      @pl.when(ci + 1 < n_chunks)
      def _():
        gather(ci + 1, 1, sem_b).wait()
        scatter(ci + 1, 1)

      return carry

    lax.fori_loop(0, -(-n_chunks // 2), body, 0)
    plsc.subcore_barrier()
    pltpu.sync_copy(acc_sh.at[pl.ds(s * rpt, rpt)],
                    out_hbm.at[c, pl.ds(s * rpt, rpt)])

  return sc_agg


def _tc1_body(pa_ref, x_ref, w1l_ref, b1_ref, w1r_ref, w2l_ref, w2r_ref,
              b2_ref, y_ref, r_ref, inv_ref, *, d):
  agg = (pa_ref[0].astype(jnp.float32)
         + pa_ref[1].astype(jnp.float32))
  cnt = agg[:, d:d + 1]
  inv = 1.0 / jnp.maximum(cnt, 1.0)
  mean = agg[:, :d] * inv
  z = lax.dot_general(mean, w1l_ref[...], (((1,), (1,)), ((), ())))
  z = z + b1_ref[...] + lax.dot_general(x_ref[...], w1r_ref[...],
                                        (((1,), (1,)), ((), ())))
  z = jnp.maximum(z, 0.0)
  y_ref[...] = lax.dot_general(z, w2l_ref[...], (((1,), (1,)), ((), ())))
  r_ref[...] = lax.dot_general(z, w2r_ref[...],
                               (((1,), (1,)), ((), ()))) + b2_ref[...]
  inv_ref[...] = jnp.broadcast_to(inv, r_ref.shape)


def _tc2_body(pb_ref, inv_ref, r_ref, out_ref):
  out_ref[...] = (pb_ref[0] + pb_ref[1]) * inv_ref[...] + r_ref[...]


def kernel(x, edge_index, W1_l, b1, W1_r, W2_l, b2, W2_r):
  n, d = x.shape
  h = W1_l.shape[0]
  out_dim = W2_l.shape[0]
  e = edge_index.shape[1]
  wext = d + 32

  k1, k2 = 128, 128

  def edge_layout(k):
    n_chunks = -(-e // (NW * k))
    e_pad = NW * k * n_chunks
    src = jnp.concatenate(
        [edge_index[0], jnp.full((e_pad - e,), n, jnp.int32)]).reshape(
            NW, n_chunks, k)
    dst = jnp.concatenate(
        [edge_index[1], jnp.full((e_pad - e,), n, jnp.int32)]).reshape(
            NW, n_chunks, k)
    return src, dst, n_chunks

  src1, dst1, n_chunks1 = edge_layout(k1)
  src2, dst2, n_chunks2 = edge_layout(k2)

  n_acc = NS * 8 * (-(-(n + 1) // (NS * 8)))

  rpt = n_acc // NS

  xe = jnp.concatenate([x.astype(jnp.bfloat16),
                        jnp.ones((n, 32), jnp.bfloat16)], axis=1)
  xe = jnp.concatenate([xe, jnp.zeros((1, wext), jnp.bfloat16)], axis=0)

  sc1 = _make_sc_agg(n + 1, wext, n_acc, n_chunks1, k1, jnp.bfloat16)
  pa = sc1(xe, src1, dst1, jnp.zeros((rpt, wext), jnp.bfloat16))

  blk = 1000
  grid = n // blk
  full = lambda shape: pl.BlockSpec(shape, lambda i: (0,) * len(shape))
  y, r, inv = pl.pallas_call(
      functools.partial(_tc1_body, d=d),
      grid=(grid,),
      in_specs=[
          pl.BlockSpec((NC, blk, wext), lambda i: (0, i, 0)),
          pl.BlockSpec((blk, d), lambda i: (i, 0)),
          full((h, d)),
          full((1, h)),
          full((h, d)),
          full((out_dim, h)),
          full((out_dim, h)),
          full((1, out_dim)),
      ],
      out_specs=[
          pl.BlockSpec((blk, out_dim), lambda i: (i, 0)),
          pl.BlockSpec((blk, out_dim), lambda i: (i, 0)),
          pl.BlockSpec((blk, out_dim), lambda i: (i, 0)),
      ],
      out_shape=[
          jax.ShapeDtypeStruct((n, out_dim), jnp.float32),
          jax.ShapeDtypeStruct((n, out_dim), jnp.float32),
          jax.ShapeDtypeStruct((n, out_dim), jnp.float32),
      ],
  )(pa, x, W1_l, b1.reshape(1, h), W1_r, W2_l, W2_r, b2.reshape(1, out_dim))

  ye = jnp.concatenate([y, jnp.zeros((1, out_dim), jnp.float32)], axis=0)
  sc2 = _make_sc_agg(n + 1, out_dim, n_acc, n_chunks2, k2, jnp.float32)
  pb = sc2(ye, src2, dst2, jnp.zeros((rpt, out_dim), jnp.float32))

  out = pl.pallas_call(
      _tc2_body,
      grid=(grid,),
      in_specs=[
          pl.BlockSpec((NC, blk, out_dim), lambda i: (0, i, 0)),
          pl.BlockSpec((blk, out_dim), lambda i: (i, 0)),
          pl.BlockSpec((blk, out_dim), lambda i: (i, 0)),
      ],
      out_specs=pl.BlockSpec((blk, out_dim), lambda i: (i, 0)),
      out_shape=jax.ShapeDtypeStruct((n, out_dim), jnp.float32),
  )(pb, inv, r)
  return out


# speedup vs baseline: 1.2831x; 1.2831x over previous
import functools

import jax
import jax.numpy as jnp
from jax import lax
from jax.experimental import pallas as pl
from jax.experimental.pallas import tpu as pltpu, tpu_sc as plsc

NS = 16
NC = 2
NW = NC * NS


def _make_sc_agg(width, n_rows_acc, n_chunks, K, dtype, tbl_in_spmem):
  rpt = n_rows_acc // NS
  mesh = plsc.VectorSubcoreMesh(core_axis_name="c", subcore_axis_name="s")

  @functools.partial(
      pl.kernel,
      out_type=jax.ShapeDtypeStruct((NC, n_rows_acc, width), dtype),
      mesh=mesh,
      compiler_params=pltpu.CompilerParams(use_tc_tiling_on_sc=False),
      scratch_types=[
          pltpu.VMEM((n_chunks, K), jnp.int32),
          pltpu.VMEM((n_chunks, K), jnp.int32),
          pltpu.VMEM((2, K, width), dtype),
          pltpu.VMEM_SHARED((n_rows_acc, width), dtype),
          pltpu.VMEM_SHARED((n_rows_acc, width), dtype) if tbl_in_spmem
          else None,
          pltpu.SemaphoreType.DMA,
          pltpu.SemaphoreType.DMA,
      ],
  )
  def sc_agg(tbl_hbm, src_hbm, dst_hbm, zeros_hbm, out_hbm,
             src_v, dst_v, rows_v, acc_sh, tbl_sh, sem_a, sem_b):
    c = lax.axis_index("c")
    s = lax.axis_index("s")
    wid = c * NS + s
    pltpu.sync_copy(zeros_hbm.at[pl.ds(s * rpt, rpt)],
                    acc_sh.at[pl.ds(s * rpt, rpt)])
    if tbl_in_spmem:
      pltpu.sync_copy(tbl_hbm.at[pl.ds(s * rpt, rpt)],
                      tbl_sh.at[pl.ds(s * rpt, rpt)])
    pltpu.sync_copy(src_hbm.at[wid], src_v)
    pltpu.sync_copy(dst_hbm.at[wid], dst_v)
    plsc.subcore_barrier()

    tbl = tbl_sh if tbl_in_spmem else tbl_hbm

    def gather(ci, buf, sem):
      return pltpu.make_async_copy(tbl.at[src_v.at[ci]],
                                   rows_v.at[buf], sem)

    def scatter(ci, buf):
      pltpu.sync_copy(rows_v.at[buf], acc_sh.at[dst_v.at[ci]], add=True)

    gather(0, 0, sem_a).start()

    def body(p, carry):
      ci = 2 * p

      @pl.when(ci + 1 < n_chunks)
      def _():
        gather(ci + 1, 1, sem_b).start()

      gather(ci, 0, sem_a).wait()
      scatter(ci, 0)

      @pl.when(ci + 2 < n_chunks)
      def _():
        gather(ci + 2, 0, sem_a).start()

      @pl.when(ci + 1 < n_chunks)
      def _():
        gather(ci + 1, 1, sem_b).wait()
        scatter(ci + 1, 1)

      return carry

    lax.fori_loop(0, -(-n_chunks // 2), body, 0)
    plsc.subcore_barrier()
    pltpu.sync_copy(acc_sh.at[pl.ds(s * rpt, rpt)],
                    out_hbm.at[c, pl.ds(s * rpt, rpt)])

  return sc_agg


def _tc1_body(pa_ref, x_ref, w1l_ref, b1_ref, w1r_ref, w2l_ref, w2r_ref,
              b2_ref, y_ref, r_ref, inv_ref, *, d):
  agg = (pa_ref[0].astype(jnp.float32)
         + pa_ref[1].astype(jnp.float32))
  cnt = agg[:, d:d + 1]
  inv = 1.0 / jnp.maximum(cnt, 1.0)
  mean = agg[:, :d] * inv
  z = lax.dot_general(mean, w1l_ref[...], (((1,), (1,)), ((), ())))
  z = z + b1_ref[...] + lax.dot_general(x_ref[...], w1r_ref[...],
                                        (((1,), (1,)), ((), ())))
  z = jnp.maximum(z, 0.0)
  y_ref[...] = lax.dot_general(z, w2l_ref[...], (((1,), (1,)), ((), ())))
  r_ref[...] = lax.dot_general(z, w2r_ref[...],
                               (((1,), (1,)), ((), ()))) + b2_ref[...]
  inv_ref[...] = jnp.broadcast_to(inv, r_ref.shape)


def _tc2_body(pb_ref, inv_ref, r_ref, out_ref):
  out_ref[...] = (pb_ref[0] + pb_ref[1]) * inv_ref[...] + r_ref[...]


def kernel(x, edge_index, W1_l, b1, W1_r, W2_l, b2, W2_r):
  n, d = x.shape
  h = W1_l.shape[0]
  out_dim = W2_l.shape[0]
  e = edge_index.shape[1]
  wext = d + 16

  k1, k2 = 64, 128

  def edge_layout(k):
    n_chunks = -(-e // (NW * k))
    e_pad = NW * k * n_chunks
    src = jnp.concatenate(
        [edge_index[0], jnp.full((e_pad - e,), n, jnp.int32)]).reshape(
            NW, n_chunks, k)
    dst = jnp.concatenate(
        [edge_index[1], jnp.full((e_pad - e,), n, jnp.int32)]).reshape(
            NW, n_chunks, k)
    return src, dst, n_chunks

  src1, dst1, n_chunks1 = edge_layout(k1)
  src2, dst2, n_chunks2 = edge_layout(k2)

  n_acc = NS * 8 * (-(-(n + 1) // (NS * 8)))

  rpt = n_acc // NS

  xe = jnp.concatenate([x, jnp.ones((n, NS), jnp.float32)], axis=1)
  xe = jnp.concatenate([xe, jnp.zeros((1, wext), jnp.float32)], axis=0)

  sc1 = _make_sc_agg(wext, n_acc, n_chunks1, k1, jnp.float32, False)
  pa = sc1(xe, src1, dst1, jnp.zeros((n_acc, wext), jnp.float32))

  blk = 1000
  grid = n // blk
  full = lambda shape: pl.BlockSpec(shape, lambda i: (0,) * len(shape))
  y, r, inv = pl.pallas_call(
      functools.partial(_tc1_body, d=d),
      grid=(grid,),
      in_specs=[
          pl.BlockSpec((NC, blk, wext), lambda i: (0, i, 0)),
          pl.BlockSpec((blk, d), lambda i: (i, 0)),
          full((h, d)),
          full((1, h)),
          full((h, d)),
          full((out_dim, h)),
          full((out_dim, h)),
          full((1, out_dim)),
      ],
      out_specs=[
          pl.BlockSpec((blk, out_dim), lambda i: (i, 0)),
          pl.BlockSpec((blk, out_dim), lambda i: (i, 0)),
          pl.BlockSpec((blk, out_dim), lambda i: (i, 0)),
      ],
      out_shape=[
          jax.ShapeDtypeStruct((n, out_dim), jnp.float32),
          jax.ShapeDtypeStruct((n, out_dim), jnp.float32),
          jax.ShapeDtypeStruct((n, out_dim), jnp.float32),
      ],
  )(pa, x, W1_l, b1.reshape(1, h), W1_r, W2_l, W2_r, b2.reshape(1, out_dim))

  ye = jnp.concatenate(
      [y, jnp.zeros((n_acc - n, out_dim), jnp.float32)], axis=0)
  sc2 = _make_sc_agg(out_dim, n_acc, n_chunks2, k2, jnp.float32, True)
  pb = sc2(ye, src2, dst2, jnp.zeros((n_acc, out_dim), jnp.float32))

  out = pl.pallas_call(
      _tc2_body,
      grid=(grid,),
      in_specs=[
          pl.BlockSpec((NC, blk, out_dim), lambda i: (0, i, 0)),
          pl.BlockSpec((blk, out_dim), lambda i: (i, 0)),
          pl.BlockSpec((blk, out_dim), lambda i: (i, 0)),
      ],
      out_specs=pl.BlockSpec((blk, out_dim), lambda i: (i, 0)),
      out_shape=jax.ShapeDtypeStruct((n, out_dim), jnp.float32),
  )(pb, inv, r)
  return out


# speedup vs baseline: 1.5537x; 1.2108x over previous
import functools

import jax
import jax.numpy as jnp
from jax import lax
from jax.experimental import pallas as pl
from jax.experimental.pallas import tpu as pltpu, tpu_sc as plsc

NS = 16
NC = 2
NW = NC * NS


def _make_sc_agg(width, n_rows_acc, n_chunks, K, dtype, tbl_in_spmem):
  rpt = n_rows_acc // NS
  mesh = plsc.VectorSubcoreMesh(core_axis_name="c", subcore_axis_name="s")

  @functools.partial(
      pl.kernel,
      out_type=jax.ShapeDtypeStruct((NC, n_rows_acc, width), dtype),
      mesh=mesh,
      compiler_params=pltpu.CompilerParams(use_tc_tiling_on_sc=False),
      scratch_types=[
          pltpu.VMEM((n_chunks, K), jnp.int32),
          pltpu.VMEM((n_chunks, K), jnp.int32),
          pltpu.VMEM((2, K, width), dtype),
          pltpu.VMEM_SHARED((n_rows_acc, width), dtype),
          pltpu.VMEM_SHARED((n_rows_acc, width), dtype) if tbl_in_spmem
          else None,
          pltpu.SemaphoreType.DMA,
          pltpu.SemaphoreType.DMA,
      ],
  )
  def sc_agg(tbl_hbm, src_hbm, dst_hbm, zeros_hbm, out_hbm,
             src_v, dst_v, rows_v, acc_sh, tbl_sh, sem_a, sem_b):
    c = lax.axis_index("c")
    s = lax.axis_index("s")
    wid = c * NS + s
    pltpu.sync_copy(zeros_hbm.at[pl.ds(s * rpt, rpt)],
                    acc_sh.at[pl.ds(s * rpt, rpt)])
    if tbl_in_spmem:
      pltpu.sync_copy(tbl_hbm.at[pl.ds(s * rpt, rpt)],
                      tbl_sh.at[pl.ds(s * rpt, rpt)])
    pltpu.sync_copy(src_hbm.at[wid], src_v)
    pltpu.sync_copy(dst_hbm.at[wid], dst_v)
    plsc.subcore_barrier()

    tbl = tbl_sh if tbl_in_spmem else tbl_hbm

    def gather(ci, buf, sem):
      return pltpu.make_async_copy(tbl.at[src_v.at[ci]],
                                   rows_v.at[buf], sem)

    def scatter(ci, buf):
      pltpu.sync_copy(rows_v.at[buf], acc_sh.at[dst_v.at[ci]], add=True)

    gather(0, 0, sem_a).start()

    def body(p, carry):
      ci = 2 * p

      @pl.when(ci + 1 < n_chunks)
      def _():
        gather(ci + 1, 1, sem_b).start()

      gather(ci, 0, sem_a).wait()
      scatter(ci, 0)

      @pl.when(ci + 2 < n_chunks)
      def _():
        gather(ci + 2, 0, sem_a).start()

      @pl.when(ci + 1 < n_chunks)
      def _():
        gather(ci + 1, 1, sem_b).wait()
        scatter(ci + 1, 1)

      return carry

    lax.fori_loop(0, -(-n_chunks // 2), body, 0)
    plsc.subcore_barrier()
    pltpu.sync_copy(acc_sh.at[pl.ds(s * rpt, rpt)],
                    out_hbm.at[c, pl.ds(s * rpt, rpt)])

  return sc_agg


def _tc1_body(pa_ref, x_ref, w1l_ref, b1_ref, w1r_ref, w2l_ref, w2r_ref,
              b2_ref, y_ref, r_ref, inv_ref, *, d):
  agg = (pa_ref[0].astype(jnp.float32)
         + pa_ref[1].astype(jnp.float32))
  cnt = agg[:, d:d + 1]
  inv = 1.0 / jnp.maximum(cnt, 1.0)
  mean = agg[:, :d] * inv
  z = lax.dot_general(mean, w1l_ref[...], (((1,), (1,)), ((), ())))
  z = z + b1_ref[...] + lax.dot_general(x_ref[...], w1r_ref[...],
                                        (((1,), (1,)), ((), ())))
  z = jnp.maximum(z, 0.0)
  y_ref[...] = lax.dot_general(z, w2l_ref[...], (((1,), (1,)), ((), ())))
  r_ref[...] = lax.dot_general(z, w2r_ref[...],
                               (((1,), (1,)), ((), ()))) + b2_ref[...]
  inv_ref[...] = jnp.broadcast_to(inv, r_ref.shape)


def _tc2_body(pb_ref, inv_ref, r_ref, out_ref):
  out_ref[...] = (pb_ref[0] + pb_ref[1]) * inv_ref[...] + r_ref[...]


def kernel(x, edge_index, W1_l, b1, W1_r, W2_l, b2, W2_r):
  n, d = x.shape
  h = W1_l.shape[0]
  out_dim = W2_l.shape[0]
  e = edge_index.shape[1]
  wext = d + 16

  k1, k2 = 64, 128

  def edge_layout(k):
    n_chunks = -(-e // (NW * k))
    e_pad = NW * k * n_chunks
    src = jnp.concatenate(
        [edge_index[0], jnp.full((e_pad - e,), n, jnp.int32)]).reshape(
            NW, n_chunks, k)
    dst = jnp.concatenate(
        [edge_index[1], jnp.full((e_pad - e,), n, jnp.int32)]).reshape(
            NW, n_chunks, k)
    return src, dst, n_chunks

  src1, dst1, n_chunks1 = edge_layout(k1)
  src2, dst2, n_chunks2 = edge_layout(k2)

  n_acc = NS * 8 * (-(-(n + 1) // (NS * 8)))

  rpt = n_acc // NS

  xe = jnp.concatenate([x.astype(jnp.bfloat16),
                        jnp.ones((n, NS), jnp.bfloat16)], axis=1)
  xe = jnp.concatenate(
      [xe, jnp.zeros((n_acc - n, wext), jnp.bfloat16)], axis=0)

  sc1 = _make_sc_agg(wext, n_acc, n_chunks1, k1, jnp.bfloat16, True)
  pa = sc1(xe, src1, dst1, jnp.zeros((n_acc, wext), jnp.bfloat16))

  blk = 1000
  grid = n // blk
  full = lambda shape: pl.BlockSpec(shape, lambda i: (0,) * len(shape))
  y, r, inv = pl.pallas_call(
      functools.partial(_tc1_body, d=d),
      grid=(grid,),
      in_specs=[
          pl.BlockSpec((NC, blk, wext), lambda i: (0, i, 0)),
          pl.BlockSpec((blk, d), lambda i: (i, 0)),
          full((h, d)),
          full((1, h)),
          full((h, d)),
          full((out_dim, h)),
          full((out_dim, h)),
          full((1, out_dim)),
      ],
      out_specs=[
          pl.BlockSpec((blk, out_dim), lambda i: (i, 0)),
          pl.BlockSpec((blk, out_dim), lambda i: (i, 0)),
          pl.BlockSpec((blk, out_dim), lambda i: (i, 0)),
      ],
      out_shape=[
          jax.ShapeDtypeStruct((n, out_dim), jnp.float32),
          jax.ShapeDtypeStruct((n, out_dim), jnp.float32),
          jax.ShapeDtypeStruct((n, out_dim), jnp.float32),
      ],
  )(pa, x, W1_l, b1.reshape(1, h), W1_r, W2_l, W2_r, b2.reshape(1, out_dim))

  ye = jnp.concatenate(
      [y, jnp.zeros((n_acc - n, out_dim), jnp.float32)], axis=0)
  sc2 = _make_sc_agg(out_dim, n_acc, n_chunks2, k2, jnp.float32, True)
  pb = sc2(ye, src2, dst2, jnp.zeros((n_acc, out_dim), jnp.float32))

  out = pl.pallas_call(
      _tc2_body,
      grid=(grid,),
      in_specs=[
          pl.BlockSpec((NC, blk, out_dim), lambda i: (0, i, 0)),
          pl.BlockSpec((blk, out_dim), lambda i: (i, 0)),
          pl.BlockSpec((blk, out_dim), lambda i: (i, 0)),
      ],
      out_specs=pl.BlockSpec((blk, out_dim), lambda i: (i, 0)),
      out_shape=jax.ShapeDtypeStruct((n, out_dim), jnp.float32),
  )(pb, inv, r)
  return out
